# Initial kernel scaffold; baseline (speedup 1.0000x reference)
import jax
import jax.numpy as jnp
from jax import lax
from jax.experimental import pallas as pl
from jax.experimental.pallas import tpu as pltpu
from jax.experimental.pallas import tpu_sc as plsc

NC = 2
NS = 16
NW = NC * NS
K = 80

_MESH = plsc.VectorSubcoreMesh(core_axis_name="c", subcore_axis_name="s")


def _sc_degree(dst3, ones_rows, zeros16, n):
    nchunks = dst3.shape[1]
    rps = n // NS

    def body(dst_hbm, ones_hbm, zeros_hbm, out_hbm, dst_v, ones_v, acc):
        cid = lax.axis_index("c")
        sid = lax.axis_index("s")
        wid = sid * NC + cid
        pltpu.sync_copy(zeros_hbm.at[pl.ds(sid * rps, rps)],
                        acc.at[pl.ds(sid * rps, rps)])
        pltpu.sync_copy(ones_hbm, ones_v)
        pltpu.sync_copy(dst_hbm.at[wid], dst_v)
        plsc.subcore_barrier()

        def step(c, carry):
            pltpu.sync_copy(ones_v, acc.at[dst_v.at[c]], add=True)
            return carry

        lax.fori_loop(0, nchunks, step, 0)
        plsc.subcore_barrier()
        pltpu.sync_copy(acc.at[pl.ds(sid * rps, rps)],
                        out_hbm.at[cid, pl.ds(sid * rps, rps)])

    f = pl.kernel(
        body,
        out_type=jax.ShapeDtypeStruct((NC, n, 16), jnp.float32),
        mesh=_MESH,
        scratch_types=[
            pltpu.VMEM((nchunks, K), jnp.int32),
            pltpu.VMEM((K, 16), jnp.float32),
            pltpu.VMEM_SHARED((n, 16), jnp.float32),
        ],
    )
    return f(dst3, ones_rows, zeros16)


def _sc_messages(g, src3, dst3, zeros, n, d):
    nchunks = src3.shape[1]
    rps = n // NS
    half = (nchunks + 1) // 2

    def body(g_hbm, src_hbm, dst_hbm, zeros_hbm, out_hbm,
             src_v, dst_v, rows, sem0, sem1, acc):
        cid = lax.axis_index("c")
        sid = lax.axis_index("s")
        wid = sid * NC + cid
        pltpu.sync_copy(zeros_hbm.at[pl.ds(sid * rps, rps)],
                        acc.at[pl.ds(sid * rps, rps)])
        pltpu.sync_copy(src_hbm.at[wid], src_v)
        pltpu.sync_copy(dst_hbm.at[wid], dst_v)
        plsc.subcore_barrier()

        pltpu.async_copy(g_hbm.at[src_v.at[0]], rows.at[0], sem0)

        def step(c2, carry):
            c0 = 2 * c2
            c1 = c0 + 1

            @pl.when(c1 < nchunks)
            def _():
                pltpu.async_copy(g_hbm.at[src_v.at[c1]], rows.at[1], sem1)

            pltpu.make_async_copy(g_hbm.at[src_v.at[c0]], rows.at[0], sem0).wait()
            pltpu.sync_copy(rows.at[0], acc.at[dst_v.at[c0]], add=True)

            @pl.when(c1 + 1 < nchunks)
            def _():
                pltpu.async_copy(g_hbm.at[src_v.at[c1 + 1]], rows.at[0], sem0)

            @pl.when(c1 < nchunks)
            def _():
                pltpu.make_async_copy(g_hbm.at[src_v.at[c1]], rows.at[1], sem1).wait()
                pltpu.sync_copy(rows.at[1], acc.at[dst_v.at[c1]], add=True)

            return carry

        lax.fori_loop(0, half, step, 0)
        plsc.subcore_barrier()
        pltpu.sync_copy(acc.at[pl.ds(sid * rps, rps)],
                        out_hbm.at[cid, pl.ds(sid * rps, rps)])

    f = pl.kernel(
        body,
        out_type=jax.ShapeDtypeStruct((NC, n, d), jnp.float32),
        mesh=_MESH,
        scratch_types=[
            pltpu.VMEM((nchunks, K), jnp.int32),
            pltpu.VMEM((nchunks, K), jnp.int32),
            pltpu.VMEM((2, K, d), jnp.float32),
            pltpu.SemaphoreType.DMA,
            pltpu.SemaphoreType.DMA,
            pltpu.VMEM_SHARED((n, d), jnp.float32),
        ],
    )
    return f(g, src3, dst3, zeros)


_ROWS = 1000


def _tc_scale_matmul(x, w, d0, d1):
    n, din = x.shape
    dh = w.shape[1]

    def body(x_ref, w_ref, d0_ref, d1_ref, g_ref):
        dinv = lax.rsqrt(d0_ref[:, :1] + d1_ref[:, :1] + 1.0)
        h = jnp.dot(x_ref[...], w_ref[...], preferred_element_type=jnp.float32)
        g_ref[...] = h * dinv

    return pl.pallas_call(
        body,
        grid=(n // _ROWS,),
        in_specs=[
            pl.BlockSpec((_ROWS, din), lambda i: (i, 0)),
            pl.BlockSpec((din, dh), lambda i: (0, 0)),
            pl.BlockSpec((_ROWS, 16), lambda i: (i, 0)),
            pl.BlockSpec((_ROWS, 16), lambda i: (i, 0)),
        ],
        out_specs=pl.BlockSpec((_ROWS, dh), lambda i: (i, 0)),
        out_shape=jax.ShapeDtypeStruct((n, dh), jnp.float32),
    )(x, w, d0, d1)


def _tc_combine_matmul(s0, s1, g, d0, d1, b, w):
    n, dh = g.shape
    dout = w.shape[1]

    def body(s0_ref, s1_ref, g_ref, d0_ref, d1_ref, b_ref, w_ref, o_ref):
        dinv = lax.rsqrt(d0_ref[:, :1] + d1_ref[:, :1] + 1.0)
        s = s0_ref[...] + s1_ref[...] + g_ref[...]
        x2 = jnp.maximum(dinv * s + b_ref[...], 0.0)
        h2 = jnp.dot(x2, w_ref[...], preferred_element_type=jnp.float32)
        o_ref[...] = h2 * dinv

    return pl.pallas_call(
        body,
        grid=(n // _ROWS,),
        in_specs=[
            pl.BlockSpec((_ROWS, dh), lambda i: (i, 0)),
            pl.BlockSpec((_ROWS, dh), lambda i: (i, 0)),
            pl.BlockSpec((_ROWS, dh), lambda i: (i, 0)),
            pl.BlockSpec((_ROWS, 16), lambda i: (i, 0)),
            pl.BlockSpec((_ROWS, 16), lambda i: (i, 0)),
            pl.BlockSpec((1, dh), lambda i: (0, 0)),
            pl.BlockSpec((dh, dout), lambda i: (0, 0)),
        ],
        out_specs=pl.BlockSpec((_ROWS, dout), lambda i: (i, 0)),
        out_shape=jax.ShapeDtypeStruct((n, dout), jnp.float32),
    )(s0, s1, g, d0, d1, b, w)


def _tc_combine(s0, s1, g, d0, d1, b):
    n, dh = g.shape

    def body(s0_ref, s1_ref, g_ref, d0_ref, d1_ref, b_ref, o_ref):
        dinv = lax.rsqrt(d0_ref[:, :1] + d1_ref[:, :1] + 1.0)
        s = s0_ref[...] + s1_ref[...] + g_ref[...]
        o_ref[...] = jnp.maximum(dinv * s + b_ref[...], 0.0)

    return pl.pallas_call(
        body,
        grid=(n // _ROWS,),
        in_specs=[
            pl.BlockSpec((_ROWS, dh), lambda i: (i, 0)),
            pl.BlockSpec((_ROWS, dh), lambda i: (i, 0)),
            pl.BlockSpec((_ROWS, dh), lambda i: (i, 0)),
            pl.BlockSpec((_ROWS, 16), lambda i: (i, 0)),
            pl.BlockSpec((_ROWS, 16), lambda i: (i, 0)),
            pl.BlockSpec((1, dh), lambda i: (0, 0)),
        ],
        out_specs=pl.BlockSpec((_ROWS, dh), lambda i: (i, 0)),
        out_shape=jax.ShapeDtypeStruct((n, dh), jnp.float32),
    )(s0, s1, g, d0, d1, b)


def kernel(x, edge_index, W1, b1, W2, b2):
    n, _ = x.shape
    e = edge_index.shape[1]
    nchunks = e // (NW * K)
    src3 = edge_index[0].reshape(NW, nchunks, K)
    dst3 = edge_index[1].reshape(NW, nchunks, K)

    ones_rows = jnp.ones((K, 16), jnp.float32)
    zeros16 = jnp.zeros((n, 16), jnp.float32)
    zeros = jnp.zeros((n, W1.shape[1]), jnp.float32)
    b1r = b1.reshape(1, -1)
    b2r = b2.reshape(1, -1)

    degp = _sc_degree(dst3, ones_rows, zeros16, n)
    d0, d1 = degp[0], degp[1]

    g1 = _tc_scale_matmul(x, W1, d0, d1)
    sp1 = _sc_messages(g1, src3, dst3, zeros, n, W1.shape[1])
    g2 = _tc_combine_matmul(sp1[0], sp1[1], g1, d0, d1, b1r, W2)
    sp2 = _sc_messages(g2, src3, dst3, zeros, n, W2.shape[1])
    return _tc_combine(sp2[0], sp2[1], g2, d0, d1, b2r)


# speedup vs baseline: 22.4808x; 22.4808x over previous
import jax
import jax.numpy as jnp
from jax import lax
from jax.experimental import pallas as pl
from jax.experimental.pallas import tpu as pltpu
from jax.experimental.pallas import tpu_sc as plsc

NC = 2
NS = 16
NW = NC * NS
K = 80

_MESH = plsc.VectorSubcoreMesh(core_axis_name="c", subcore_axis_name="s")
_SC_PARAMS = pltpu.CompilerParams(use_tc_tiling_on_sc=False)


def _sc_degree(dst3, ones_rows, zeros16, n):
    nchunks = dst3.shape[1]
    rps = n // NS

    def body(dst_hbm, ones_hbm, zeros_hbm, out_hbm, dst_v, ones_v, acc):
        cid = lax.axis_index("c")
        sid = lax.axis_index("s")
        wid = sid * NC + cid
        pltpu.sync_copy(zeros_hbm.at[sid], acc.at[pl.ds(sid * rps, rps)])
        pltpu.sync_copy(ones_hbm, ones_v)
        pltpu.sync_copy(dst_hbm.at[wid], dst_v)
        plsc.subcore_barrier()

        def step(c, carry):
            pltpu.sync_copy(ones_v, acc.at[dst_v.at[c]], add=True)
            return carry

        lax.fori_loop(0, nchunks, step, 0)
        plsc.subcore_barrier()
        pltpu.sync_copy(acc.at[pl.ds(sid * rps, rps)], out_hbm.at[cid, sid])

    f = pl.kernel(
        body,
        out_type=jax.ShapeDtypeStruct((NC, NS, rps, 16), jnp.float32),
        mesh=_MESH,
        compiler_params=_SC_PARAMS,
        scratch_types=[
            pltpu.VMEM((nchunks, K), jnp.int32),
            pltpu.VMEM((K, 16), jnp.float32),
            pltpu.VMEM_SHARED((n, 16), jnp.float32),
        ],
    )
    return f(dst3, ones_rows, zeros16)


def _sc_messages(g_stacked, src3, dst3, zeros, n, dhalf):
    nchunks = src3.shape[1]
    rps = n // NS
    half = (nchunks + 1) // 2

    def body(g_hbm, src_hbm, dst_hbm, zeros_hbm, out_hbm,
             src_v, dst_v, rows, sem0, sem1, acc):
        cid = lax.axis_index("c")
        sid = lax.axis_index("s")
        table = g_hbm.at[cid]
        pltpu.sync_copy(zeros_hbm.at[sid], acc.at[pl.ds(sid * rps, rps)])
        pltpu.sync_copy(src_hbm.at[sid], src_v)
        pltpu.sync_copy(dst_hbm.at[sid], dst_v)
        plsc.subcore_barrier()

        pltpu.async_copy(table.at[src_v.at[0]], rows.at[0], sem0)

        def step(c2, carry):
            c0 = 2 * c2
            c1 = c0 + 1

            @pl.when(c1 < nchunks)
            def _():
                pltpu.async_copy(table.at[src_v.at[c1]], rows.at[1], sem1)

            pltpu.make_async_copy(table.at[src_v.at[c0]], rows.at[0], sem0).wait()
            pltpu.sync_copy(rows.at[0], acc.at[dst_v.at[c0]], add=True)

            @pl.when(c1 + 1 < nchunks)
            def _():
                pltpu.async_copy(table.at[src_v.at[c1 + 1]], rows.at[0], sem0)

            @pl.when(c1 < nchunks)
            def _():
                pltpu.make_async_copy(table.at[src_v.at[c1]], rows.at[1], sem1).wait()
                pltpu.sync_copy(rows.at[1], acc.at[dst_v.at[c1]], add=True)

            return carry

        lax.fori_loop(0, half, step, 0)
        plsc.subcore_barrier()
        pltpu.sync_copy(acc.at[pl.ds(sid * rps, rps)], out_hbm.at[cid, sid])

    f = pl.kernel(
        body,
        out_type=jax.ShapeDtypeStruct((NC, NS, rps, dhalf), jnp.float32),
        mesh=_MESH,
        compiler_params=_SC_PARAMS,
        scratch_types=[
            pltpu.VMEM((nchunks, K), jnp.int32),
            pltpu.VMEM((nchunks, K), jnp.int32),
            pltpu.VMEM((2, K, dhalf), jnp.float32),
            pltpu.SemaphoreType.DMA,
            pltpu.SemaphoreType.DMA,
            pltpu.VMEM_SHARED((n, dhalf), jnp.float32),
        ],
    )
    return f(g_stacked, src3, dst3, zeros)


_ROWS = 1000


def _deg_specs(two):
    return [
        pl.BlockSpec((1, _ROWS, 16), lambda i: (0, i, 0)),
        pl.BlockSpec((1, _ROWS, 16), lambda i: (1, i, 0)),
    ] if two else []


def _tc_scale_matmul(x, w, degp):
    n, din = x.shape
    dh = w.shape[1]

    def body(x_ref, w_ref, dp0_ref, dp1_ref, g_ref):
        dinv = lax.rsqrt(dp0_ref[0][:, :1] + dp1_ref[0][:, :1] + 1.0)
        h = jnp.dot(x_ref[...], w_ref[...], preferred_element_type=jnp.float32)
        g_ref[...] = h * dinv

    return pl.pallas_call(
        body,
        grid=(n // _ROWS,),
        in_specs=[
            pl.BlockSpec((_ROWS, din), lambda i: (i, 0)),
            pl.BlockSpec((din, dh), lambda i: (0, 0)),
        ] + _deg_specs(True),
        out_specs=pl.BlockSpec((_ROWS, dh), lambda i: (i, 0)),
        out_shape=jax.ShapeDtypeStruct((n, dh), jnp.float32),
    )(x, w, degp, degp)


def _tc_combine_matmul(sp, g, degp, b, w):
    n, dh = g.shape
    dhalf = sp.shape[-1]
    dout = w.shape[1]

    def body(s0_ref, s1_ref, g_ref, dp0_ref, dp1_ref, b_ref, w_ref, o_ref):
        dinv = lax.rsqrt(dp0_ref[0][:, :1] + dp1_ref[0][:, :1] + 1.0)
        s = jnp.concatenate([s0_ref[0], s1_ref[0]], axis=-1) + g_ref[...]
        x2 = jnp.maximum(dinv * s + b_ref[...], 0.0)
        h2 = jnp.dot(x2, w_ref[...], preferred_element_type=jnp.float32)
        o_ref[...] = h2 * dinv

    return pl.pallas_call(
        body,
        grid=(n // _ROWS,),
        in_specs=[
            pl.BlockSpec((1, _ROWS, dhalf), lambda i: (0, i, 0)),
            pl.BlockSpec((1, _ROWS, dhalf), lambda i: (1, i, 0)),
            pl.BlockSpec((_ROWS, dh), lambda i: (i, 0)),
        ] + _deg_specs(True) + [
            pl.BlockSpec((1, dh), lambda i: (0, 0)),
            pl.BlockSpec((dh, dout), lambda i: (0, 0)),
        ],
        out_specs=pl.BlockSpec((_ROWS, dout), lambda i: (i, 0)),
        out_shape=jax.ShapeDtypeStruct((n, dout), jnp.float32),
    )(sp, sp, g, degp, degp, b, w)


def _tc_combine(sp, g, degp, b):
    n, dh = g.shape
    dhalf = sp.shape[-1]

    def body(s0_ref, s1_ref, g_ref, dp0_ref, dp1_ref, b_ref, o_ref):
        dinv = lax.rsqrt(dp0_ref[0][:, :1] + dp1_ref[0][:, :1] + 1.0)
        s = jnp.concatenate([s0_ref[0], s1_ref[0]], axis=-1) + g_ref[...]
        o_ref[...] = jnp.maximum(dinv * s + b_ref[...], 0.0)

    return pl.pallas_call(
        body,
        grid=(n // _ROWS,),
        in_specs=[
            pl.BlockSpec((1, _ROWS, dhalf), lambda i: (0, i, 0)),
            pl.BlockSpec((1, _ROWS, dhalf), lambda i: (1, i, 0)),
            pl.BlockSpec((_ROWS, dh), lambda i: (i, 0)),
        ] + _deg_specs(True) + [
            pl.BlockSpec((1, dh), lambda i: (0, 0)),
        ],
        out_specs=pl.BlockSpec((_ROWS, dh), lambda i: (i, 0)),
        out_shape=jax.ShapeDtypeStruct((n, dh), jnp.float32),
    )(sp, sp, g, degp, degp, b)


def kernel(x, edge_index, W1, b1, W2, b2):
    n, _ = x.shape
    e = edge_index.shape[1]
    dh = W1.shape[1]
    dhalf = dh // NC
    rps = n // NS

    c_deg = e // (NW * K)
    dst3d = edge_index[1].reshape(NW, c_deg, K)
    c_msg = e // (NS * K)
    src3 = edge_index[0].reshape(NS, c_msg, K)
    dst3 = edge_index[1].reshape(NS, c_msg, K)

    ones_rows = jnp.ones((K, 16), jnp.float32)
    zeros16 = jnp.zeros((NS, rps, 16), jnp.float32)
    zeros = jnp.zeros((NS, rps, dhalf), jnp.float32)
    b1r = b1.reshape(1, -1)
    b2r = b2.reshape(1, -1)

    degp = _sc_degree(dst3d, ones_rows, zeros16, n).reshape(NC, n, 16)

    g1 = _tc_scale_matmul(x, W1, degp)
    g1s = jnp.stack([g1[:, :dhalf], g1[:, dhalf:]])
    sp1 = _sc_messages(g1s, src3, dst3, zeros, n, dhalf).reshape(NC, n, dhalf)
    g2 = _tc_combine_matmul(sp1, g1, degp, b1r, W2)
    g2s = jnp.stack([g2[:, :dhalf], g2[:, dhalf:]])
    sp2 = _sc_messages(g2s, src3, dst3, zeros, n, dhalf).reshape(NC, n, dhalf)
    return _tc_combine(sp2, g2, degp, b2r)


# speedup vs baseline: 30.0005x; 1.3345x over previous
import jax
import jax.numpy as jnp
from jax import lax
from jax.experimental import pallas as pl
from jax.experimental.pallas import tpu as pltpu
from jax.experimental.pallas import tpu_sc as plsc

NC = 2
NS = 16
NW = NC * NS
K = 80
KM = 125
NB = 4

_MESH = plsc.VectorSubcoreMesh(core_axis_name="c", subcore_axis_name="s")
_SC_PARAMS = pltpu.CompilerParams(use_tc_tiling_on_sc=False)


def _sc_degree(dst3, ones_rows, zeros16, n):
    nchunks = dst3.shape[1]
    rps = n // NS

    def body(dst_hbm, ones_hbm, zeros_hbm, out_hbm, dst_v, ones_v, acc):
        cid = lax.axis_index("c")
        sid = lax.axis_index("s")
        wid = sid * NC + cid
        pltpu.sync_copy(zeros_hbm.at[sid], acc.at[pl.ds(sid * rps, rps)])
        pltpu.sync_copy(ones_hbm, ones_v)
        pltpu.sync_copy(dst_hbm.at[wid], dst_v)
        plsc.subcore_barrier()

        def step(c, carry):
            pltpu.sync_copy(ones_v, acc.at[dst_v.at[c]], add=True)
            return carry

        lax.fori_loop(0, nchunks, step, 0)
        plsc.subcore_barrier()
        pltpu.sync_copy(acc.at[pl.ds(sid * rps, rps)], out_hbm.at[cid, sid])

    f = pl.kernel(
        body,
        out_type=jax.ShapeDtypeStruct((NC, NS, rps, 16), jnp.float32),
        mesh=_MESH,
        compiler_params=_SC_PARAMS,
        scratch_types=[
            pltpu.VMEM((nchunks, K), jnp.int32),
            pltpu.VMEM((K, 16), jnp.float32),
            pltpu.VMEM_SHARED((n, 16), jnp.float32),
        ],
    )
    return f(dst3, ones_rows, zeros16)


def _sc_messages(g_stacked, src3, dst3, zeros, n, dhalf):
    nchunks = src3.shape[1]
    rps = n // NS
    assert nchunks % NB == 0

    def body(g_hbm, src_hbm, dst_hbm, zeros_hbm, out_hbm,
             src_v, dst_v, rows, gsems, ssems, acc):
        cid = lax.axis_index("c")
        sid = lax.axis_index("s")
        table = g_hbm.at[cid]
        pltpu.sync_copy(zeros_hbm.at[sid], acc.at[pl.ds(sid * rps, rps)])
        pltpu.sync_copy(src_hbm.at[sid], src_v)
        pltpu.sync_copy(dst_hbm.at[sid], dst_v)
        plsc.subcore_barrier()

        def gather(c, b):
            pltpu.async_copy(table.at[src_v.at[c]], rows.at[b], gsems[b])

        def wait_gather(c, b):
            pltpu.make_async_copy(table.at[src_v.at[c]], rows.at[b],
                                  gsems[b]).wait()

        def scatter(c, b):
            pltpu.async_copy(rows.at[b], acc.at[dst_v.at[c]], ssems[b],
                             add=True)

        def wait_scatter(c, b):
            pltpu.make_async_copy(rows.at[b], acc.at[dst_v.at[c]],
                                  ssems[b]).wait()

        gather(0, 0)
        gather(1, 1)

        def step(c4, carry):
            for j in range(NB):
                c = NB * c4 + j
                b2 = (j + 2) % NB

                @pl.when(c + 2 < nchunks)
                def _():
                    @pl.when(c >= 2)
                    def _():
                        wait_scatter(c - 2, b2)
                    gather(c + 2, b2)

                wait_gather(c, j)
                scatter(c, j)
            return carry

        lax.fori_loop(0, nchunks // NB, step, 0)
        for j in range(NB):
            wait_scatter(nchunks - NB + j, j)
        plsc.subcore_barrier()
        pltpu.sync_copy(acc.at[pl.ds(sid * rps, rps)], out_hbm.at[cid, sid])

    f = pl.kernel(
        body,
        out_type=jax.ShapeDtypeStruct((NC, NS, rps, dhalf), jnp.float32),
        mesh=_MESH,
        compiler_params=_SC_PARAMS,
        scratch_types=[
            pltpu.VMEM((nchunks, KM), jnp.int32),
            pltpu.VMEM((nchunks, KM), jnp.int32),
            pltpu.VMEM((NB, KM, dhalf), jnp.float32),
            [pltpu.SemaphoreType.DMA] * NB,
            [pltpu.SemaphoreType.DMA] * NB,
            pltpu.VMEM_SHARED((n, dhalf), jnp.float32),
        ],
    )
    return f(g_stacked, src3, dst3, zeros)


_ROWS = 1000


def _deg_specs(two):
    return [
        pl.BlockSpec((1, _ROWS, 16), lambda i: (0, i, 0)),
        pl.BlockSpec((1, _ROWS, 16), lambda i: (1, i, 0)),
    ] if two else []


def _half_specs(dhalf):
    return [
        pl.BlockSpec((1, _ROWS, dhalf), lambda i: (0, i, 0)),
        pl.BlockSpec((1, _ROWS, dhalf), lambda i: (1, i, 0)),
    ]


def _tc_scale_matmul(x, w, degp):
    n, din = x.shape
    dh = w.shape[1]
    dhalf = dh // NC

    def body(x_ref, w_ref, dp0_ref, dp1_ref, g_ref):
        dinv = lax.rsqrt(dp0_ref[0][:, :1] + dp1_ref[0][:, :1] + 1.0)
        h = jnp.dot(x_ref[...], w_ref[...], preferred_element_type=jnp.float32)
        g = h * dinv
        g_ref[0] = g[:, :dhalf]
        g_ref[1] = g[:, dhalf:]

    return pl.pallas_call(
        body,
        grid=(n // _ROWS,),
        in_specs=[
            pl.BlockSpec((_ROWS, din), lambda i: (i, 0)),
            pl.BlockSpec((din, dh), lambda i: (0, 0)),
        ] + _deg_specs(True),
        out_specs=pl.BlockSpec((NC, _ROWS, dhalf), lambda i: (0, i, 0)),
        out_shape=jax.ShapeDtypeStruct((NC, n, dhalf), jnp.float32),
    )(x, w, degp, degp)


def _tc_combine_matmul(sp, gs, degp, b, w):
    _, n, dhalf = gs.shape
    dh = NC * dhalf
    dout = w.shape[1]

    def body(s0_ref, s1_ref, g0_ref, g1_ref, dp0_ref, dp1_ref, b_ref, w_ref,
             o_ref):
        dinv = lax.rsqrt(dp0_ref[0][:, :1] + dp1_ref[0][:, :1] + 1.0)
        s = jnp.concatenate([s0_ref[0] + g0_ref[0], s1_ref[0] + g1_ref[0]],
                            axis=-1)
        x2 = jnp.maximum(dinv * s + b_ref[...], 0.0)
        h2 = jnp.dot(x2, w_ref[...], preferred_element_type=jnp.float32)
        g = h2 * dinv
        o_ref[0] = g[:, :dhalf]
        o_ref[1] = g[:, dhalf:]

    return pl.pallas_call(
        body,
        grid=(n // _ROWS,),
        in_specs=_half_specs(dhalf) + _half_specs(dhalf) + _deg_specs(True) + [
            pl.BlockSpec((1, dh), lambda i: (0, 0)),
            pl.BlockSpec((dh, dout), lambda i: (0, 0)),
        ],
        out_specs=pl.BlockSpec((NC, _ROWS, dhalf), lambda i: (0, i, 0)),
        out_shape=jax.ShapeDtypeStruct((NC, n, dhalf), jnp.float32),
    )(sp, sp, gs, gs, degp, degp, b, w)


def _tc_combine(sp, gs, degp, b):
    _, n, dhalf = gs.shape
    dh = NC * dhalf

    def body(s0_ref, s1_ref, g0_ref, g1_ref, dp0_ref, dp1_ref, b_ref, o_ref):
        dinv = lax.rsqrt(dp0_ref[0][:, :1] + dp1_ref[0][:, :1] + 1.0)
        s = jnp.concatenate([s0_ref[0] + g0_ref[0], s1_ref[0] + g1_ref[0]],
                            axis=-1)
        o_ref[...] = jnp.maximum(dinv * s + b_ref[...], 0.0)

    return pl.pallas_call(
        body,
        grid=(n // _ROWS,),
        in_specs=_half_specs(dhalf) + _half_specs(dhalf) + _deg_specs(True) + [
            pl.BlockSpec((1, dh), lambda i: (0, 0)),
        ],
        out_specs=pl.BlockSpec((_ROWS, dh), lambda i: (i, 0)),
        out_shape=jax.ShapeDtypeStruct((n, dh), jnp.float32),
    )(sp, sp, gs, gs, degp, degp, b)


def kernel(x, edge_index, W1, b1, W2, b2):
    n, _ = x.shape
    e = edge_index.shape[1]
    dh = W1.shape[1]
    dhalf = dh // NC
    rps = n // NS

    c_deg = e // (NW * K)
    dst3d = edge_index[1].reshape(NW, c_deg, K)
    c_msg = e // (NS * KM)
    src3 = edge_index[0].reshape(NS, c_msg, KM)
    dst3 = edge_index[1].reshape(NS, c_msg, KM)

    ones_rows = jnp.ones((K, 16), jnp.float32)
    zeros16 = jnp.zeros((NS, rps, 16), jnp.float32)
    zeros = jnp.zeros((NS, rps, dhalf), jnp.float32)
    b1r = b1.reshape(1, -1)
    b2r = b2.reshape(1, -1)

    degp = _sc_degree(dst3d, ones_rows, zeros16, n).reshape(NC, n, 16)

    g1s = _tc_scale_matmul(x, W1, degp)
    sp1 = _sc_messages(g1s, src3, dst3, zeros, n, dhalf).reshape(NC, n, dhalf)
    g2s = _tc_combine_matmul(sp1, g1s, degp, b1r, W2)
    sp2 = _sc_messages(g2s, src3, dst3, zeros, n, dhalf).reshape(NC, n, dhalf)
    return _tc_combine(sp2, g2s, degp, b2r)
